# baseline (speedup 1.0000x reference)
import jax
import jax.numpy as jnp
from jax import lax
from jax.experimental import pallas as pl
from jax.experimental.pallas import tpu as pltpu

_BLOCK_ROWS = 25000


def _prompt_block_kernel(x_ref, ts_ref, to_ref, o_ref):
    x_blk = x_ref[...]
    logits2 = lax.dot_general(
        x_blk, ts_ref[...], (((1,), (1,)), ((), ())),
        preferred_element_type=jnp.float32)
    e = jnp.exp2(logits2)
    pd = lax.dot_general(
        e, to_ref[...], (((1,), (0,)), ((), ())),
        preferred_element_type=jnp.float32)
    o_ref[...] = x_blk + pd[:, :128] / pd[:, 128:]


def kernel(x, token_embeds):
    n, d = x.shape
    t_num = token_embeds.shape[0]
    t_scaled = token_embeds * jnp.float32(1.4426950408889634)
    t_ones = jnp.concatenate(
        [token_embeds, jnp.ones((t_num, d), jnp.float32)], axis=1)
    bn = _BLOCK_ROWS
    grid = (pl.cdiv(n, bn),)
    return pl.pallas_call(
        _prompt_block_kernel,
        grid=grid,
        in_specs=[
            pl.BlockSpec((bn, d), lambda i: (i, 0)),
            pl.BlockSpec((t_num, d), lambda i: (0, 0)),
            pl.BlockSpec((t_num, 2 * d), lambda i: (0, 0)),
        ],
        out_specs=pl.BlockSpec((bn, d), lambda i: (i, 0)),
        out_shape=jax.ShapeDtypeStruct((n, d), x.dtype),
        compiler_params=pltpu.CompilerParams(
            dimension_semantics=("parallel",)),
    )(x, t_scaled, t_ones)


# speedup vs baseline: 1.0625x; 1.0625x over previous
import jax
import jax.numpy as jnp
from jax import lax
from jax.experimental import pallas as pl
from jax.experimental.pallas import tpu as pltpu

_BLOCK_ROWS = 25000


def _prompt_block_kernel(x_ref, t_ref, o_ref):
    x_blk = x_ref[...]
    t = t_ref[...]
    logits = lax.dot_general(
        x_blk, t, (((1,), (1,)), ((), ())),
        preferred_element_type=jnp.float32)
    m = jnp.max(logits, axis=1, keepdims=True)
    e = jnp.exp(logits - m)
    attn = e / jnp.sum(e, axis=1, keepdims=True)
    prompt = lax.dot_general(
        attn, t, (((1,), (0,)), ((), ())),
        preferred_element_type=jnp.float32)
    o_ref[...] = x_blk + prompt


def kernel(x, token_embeds):
    n, d = x.shape
    t_num = token_embeds.shape[0]
    bn = _BLOCK_ROWS
    grid = (pl.cdiv(n, bn),)
    return pl.pallas_call(
        _prompt_block_kernel,
        grid=grid,
        in_specs=[
            pl.BlockSpec((bn, d), lambda i: (i, 0)),
            pl.BlockSpec((t_num, d), lambda i: (0, 0)),
        ],
        out_specs=pl.BlockSpec((bn, d), lambda i: (i, 0)),
        out_shape=jax.ShapeDtypeStruct((n, d), x.dtype),
        compiler_params=pltpu.CompilerParams(
            dimension_semantics=("parallel",)),
    )(x, token_embeds)
